# baseline (speedup 1.0000x reference)
import functools

import jax
import jax.numpy as jnp
from jax import lax
from jax.experimental import pallas as pl
from jax.experimental.pallas import tpu as pltpu
from jax.experimental.pallas import tpu_sc as plsc

N = 2000
NP = 2048
EP = 32768
NW = 32
EPW = EP // NW
NDMA = 2 * EPW // 128


WROWS = 512
WINW = WROWS * NP
NWIN = 2
OSL = WINW // 16
EPS = EP // 16
ZN = 16384


def _sc_build_body(pk_hbm, b_out, shared, pk_v, idx_v, vals_v, zbuf, sem):
    c = lax.axis_index("c")
    sid = lax.axis_index("s")

    @pl.loop(0, ZN // 16)
    def _zero(i):
        zbuf[pl.ds(i * 16, 16)] = jnp.zeros((16,), jnp.float32)

    for k in range(8):
        vals_v[pl.ds(k * 16, 16)] = jnp.full((16,), 1.0, jnp.float32)
    zcopies = [
        pltpu.make_async_copy(zbuf, shared.at[pl.ds(sid * OSL + k * ZN, ZN)],
                              sem)
        for k in range(OSL // ZN)
    ]
    for cp in zcopies:
        cp.start()
    pltpu.sync_copy(pk_hbm.at[pl.ds(sid * EPS, EPS)], pk_v)
    for cp in zcopies:
        cp.wait()

    for w in range(NWIN):
        base = (c * NWIN + w) * WINW
        for i in range(EPS // 16):
            p = pk_v[pl.ds(i * 16, 16)]
            s = lax.shift_right_logical(p, 11)
            d = p & (NP - 1)
            for k, a in enumerate((p, d * NP + s)):
                loc = a - base
                ok = (loc >= 0) & (loc < WINW)
                r, col = divmod((k * EPS + i * 16), 128)
                idx_v[r, pl.ds(col, 16)] = jnp.where(ok, loc, NP - 2)
        plsc.subcore_barrier()
        copies = [
            pltpu.make_async_copy(vals_v, shared.at[idx_v.at[j]], sem)
            for j in range(2 * EPS // 128)
        ]
        for cp in copies:
            cp.start()
        for cp in copies:
            cp.wait()
        plsc.subcore_barrier()
        pltpu.sync_copy(shared.at[pl.ds(sid * OSL, OSL)],
                        b_out.at[pl.ds(base + sid * OSL, OSL)])
        if w + 1 < NWIN:
            plsc.subcore_barrier()
            undo = [
                pltpu.make_async_copy(zbuf.at[pl.ds(0, 128)],
                                      shared.at[idx_v.at[j]], sem)
                for j in range(2 * EPS // 128)
            ]
            for cp in undo:
                cp.start()
            for cp in undo:
                cp.wait()


def _build_adjacency(edges):
    pad = jnp.full((EP - edges.shape[1],), NP - 1, jnp.int32)
    src = jnp.concatenate([edges[0], pad])
    dst = jnp.concatenate([edges[1], pad])
    pk = src * NP + dst
    mesh = plsc.VectorSubcoreMesh(core_axis_name="c", subcore_axis_name="s")
    sc = pl.kernel(
        _sc_build_body,
        out_type=jax.ShapeDtypeStruct((NP * NP,), jnp.float32),
        mesh=mesh,
        scratch_types=[
            pltpu.VMEM_SHARED((WINW,), jnp.float32),
            pltpu.VMEM((EPS,), jnp.int32),
            pltpu.VMEM((2 * EPS // 128, 128), jnp.int32),
            pltpu.VMEM((128,), jnp.float32),
            pltpu.VMEM((ZN,), jnp.float32),
            pltpu.SemaphoreType.DMA,
        ],
    )
    return sc(pk).reshape(NP, NP)


def _mlp_body(xp_ref, xf_ref, wp1, bp1, wp2, bp2, wp3, bp3, wf1, bf1, wf2, bf2,
              h_ref):
    dot = functools.partial(jnp.dot, preferred_element_type=jnp.float32)
    h = jax.nn.relu(dot(xp_ref[...], wp1[...]) + bp1[...])
    h = jax.nn.relu(dot(h, wp2[...]) + bp2[...])
    hp = jax.nn.relu(dot(h, wp3[...]) + bp3[...])
    g = jax.nn.relu(dot(xf_ref[...], wf1[...]) + bf1[...])
    hf = jax.nn.relu(dot(g, wf2[...]) + bf2[...])
    rows = lax.broadcasted_iota(jnp.int32, (NP, 1), 0)
    valid = (rows < N).astype(jnp.float32)
    h_ref[:, 0:1] = hp * valid
    h_ref[:, 1:2] = hf * valid
    h_ref[:, 2:3] = valid


def _mlp(x_part, x_family, wp1, bp1, wp2, bp2, wp3, bp3, wf1, bf1, wf2, bf2):
    xp = jnp.zeros((NP, x_part.shape[1]), jnp.float32).at[:N].set(x_part)
    xf = jnp.zeros((NP, x_family.shape[1]), jnp.float32).at[:N].set(x_family)
    return pl.pallas_call(
        _mlp_body,
        out_shape=jax.ShapeDtypeStruct((NP, 3), jnp.float32),
    )(xp, xf, wp1, bp1.reshape(1, -1), wp2, bp2.reshape(1, -1), wp3,
      bp3.reshape(1, -1), wf1, bf1.reshape(1, -1), wf2, bf2.reshape(1, -1))


BR = 256
GRID = NP // BR


def _rgcn_body(din, dout, b_ref, he_ref, wrel0, wrel1, wroot, bias, out_ref):
    i = pl.program_id(0)
    dot = functools.partial(jnp.dot, preferred_element_type=jnp.float32)
    rows = i * BR + lax.broadcasted_iota(jnp.int32, (BR, NP), 0)
    cols = lax.broadcasted_iota(jnp.int32, (BR, NP), 1)
    bblk = jnp.where(rows == cols, 0.0, b_ref[...])
    se = dot(bblk, he_ref[...])
    s1 = se[:, 0:din]
    deg = se[:, din:din + 1]
    hblk = he_ref[pl.ds(i * BR, BR), 0:din]
    total = jnp.sum(he_ref[:, 0:din], axis=0, keepdims=True)
    agg0 = total - hblk - s1
    cnt0 = jnp.maximum((N - 1.0) - deg, 1.0)
    cnt1 = jnp.maximum(deg, 1.0)
    out = (dot(hblk, wroot[...]) + bias[...]
           + dot(agg0 / cnt0, wrel0[...])
           + dot(s1 / cnt1, wrel1[...]))
    out = jax.nn.relu(out)
    rmask = (i * BR + lax.broadcasted_iota(jnp.int32, (BR, 1), 0)) < N
    out = jnp.where(rmask, out, 0.0)
    out_ref[:, 0:dout] = out
    if dout < out_ref.shape[1]:
        out_ref[:, dout:dout + 1] = rmask.astype(jnp.float32)


def _rgcn_layer(bmat, he, din, dout, wrel, wroot, bias, with_ones):
    owid = dout + 1 if with_ones else dout
    full = lambda a: pl.BlockSpec(a.shape, lambda i: (0,) * a.ndim)
    return pl.pallas_call(
        functools.partial(_rgcn_body, din, dout),
        grid=(GRID,),
        in_specs=[
            pl.BlockSpec((BR, NP), lambda i: (i, 0)),
            full(he),
            pl.BlockSpec((din, dout), lambda i: (0, 0)),
            pl.BlockSpec((din, dout), lambda i: (0, 0)),
            pl.BlockSpec((din, dout), lambda i: (0, 0)),
            pl.BlockSpec((1, dout), lambda i: (0, 0)),
        ],
        out_specs=pl.BlockSpec((BR, owid), lambda i: (i, 0)),
        out_shape=jax.ShapeDtypeStruct((NP, owid), jnp.float32),
    )(bmat, he, wrel[0], wrel[1], wroot, bias.reshape(1, -1))


def kernel(x_part, x_family, edges, Wp1, bp1, Wp2, bp2, Wp3, bp3,
           Wf1, bf1, Wf2, bf2, Wrel1, Wroot1, b1, Wrel2, Wroot2, b2):
    bmat = _build_adjacency(edges)
    h1e = _mlp(x_part, x_family, Wp1, bp1, Wp2, bp2, Wp3, bp3,
               Wf1, bf1, Wf2, bf2)
    h2e = _rgcn_layer(bmat, h1e, 2, 8, Wrel1, Wroot1, b1, with_ones=True)
    h3 = _rgcn_layer(bmat, h2e, 8, 16, Wrel2, Wroot2, b2, with_ones=False)
    return h3[:N]


# speedup vs baseline: 1.1788x; 1.1788x over previous
import functools

import jax
import jax.numpy as jnp
from jax import lax
from jax.experimental import pallas as pl
from jax.experimental.pallas import tpu as pltpu
from jax.experimental.pallas import tpu_sc as plsc

N = 2000
NP = 2048
EP = 32768
NW = 32
EPW = EP // NW
NDMA = 2 * EPW // 128


WROWS = 512
WINW = WROWS * NP
NWIN = 2
OSL = WINW // 16
EPS = EP // 16
ZN = 16384


def _sc_build_body(pk_hbm, b_out, shared, pk_v, idx_v, vals_v, zbuf, sem):
    c = lax.axis_index("c")
    sid = lax.axis_index("s")

    @pl.loop(0, ZN // 16)
    def _zero(i):
        zbuf[pl.ds(i * 16, 16)] = jnp.zeros((16,), jnp.float32)

    for k in range(8):
        vals_v[pl.ds(k * 16, 16)] = jnp.full((16,), 1.0, jnp.float32)
    zcopies = [
        pltpu.make_async_copy(zbuf, shared.at[pl.ds(sid * OSL + k * ZN, ZN)],
                              sem)
        for k in range(OSL // ZN)
    ]
    for cp in zcopies:
        cp.start()
    pltpu.sync_copy(pk_hbm.at[pl.ds(sid * EPS, EPS)], pk_v)
    for cp in zcopies:
        cp.wait()

    for w in range(NWIN):
        base = (c * NWIN + w) * WINW
        for i in range(EPS // 16):
            p = pk_v[pl.ds(i * 16, 16)]
            s = lax.shift_right_logical(p, 11)
            d = p & (NP - 1)
            for k, a in enumerate((p, d * NP + s)):
                loc = a - base
                ok = (loc >= 0) & (loc < WINW)
                r, col = divmod((k * EPS + i * 16), 128)
                idx_v[r, pl.ds(col, 16)] = jnp.where(ok, loc, NP - 2)
        plsc.subcore_barrier()
        copies = [
            pltpu.make_async_copy(vals_v, shared.at[idx_v.at[j]], sem)
            for j in range(2 * EPS // 128)
        ]
        for cp in copies:
            cp.start()
        for cp in copies:
            cp.wait()
        plsc.subcore_barrier()
        pltpu.sync_copy(shared.at[pl.ds(sid * OSL, OSL)],
                        b_out.at[pl.ds(base + sid * OSL, OSL)])
        if w + 1 < NWIN:
            rz = [
                pltpu.make_async_copy(
                    zbuf, shared.at[pl.ds(sid * OSL + k * ZN, ZN)], sem)
                for k in range(OSL // ZN)
            ]
            for cp in rz:
                cp.start()
            for cp in rz:
                cp.wait()


def _build_adjacency(edges):
    pad = jnp.full((EP - edges.shape[1],), NP - 1, jnp.int32)
    src = jnp.concatenate([edges[0], pad])
    dst = jnp.concatenate([edges[1], pad])
    pk = src * NP + dst
    mesh = plsc.VectorSubcoreMesh(core_axis_name="c", subcore_axis_name="s")
    sc = pl.kernel(
        _sc_build_body,
        out_type=jax.ShapeDtypeStruct((NP * NP,), jnp.float32),
        mesh=mesh,
        scratch_types=[
            pltpu.VMEM_SHARED((WINW,), jnp.float32),
            pltpu.VMEM((EPS,), jnp.int32),
            pltpu.VMEM((2 * EPS // 128, 128), jnp.int32),
            pltpu.VMEM((128,), jnp.float32),
            pltpu.VMEM((ZN,), jnp.float32),
            pltpu.SemaphoreType.DMA,
        ],
    )
    return sc(pk).reshape(NP, NP)


def _mlp_body(xp_ref, xf_ref, wp1, bp1, wp2, bp2, wp3, bp3, wf1, bf1, wf2, bf2,
              h_ref):
    dot = functools.partial(jnp.dot, preferred_element_type=jnp.float32)
    h = jax.nn.relu(dot(xp_ref[...], wp1[...]) + bp1[...])
    h = jax.nn.relu(dot(h, wp2[...]) + bp2[...])
    hp = jax.nn.relu(dot(h, wp3[...]) + bp3[...])
    g = jax.nn.relu(dot(xf_ref[...], wf1[...]) + bf1[...])
    hf = jax.nn.relu(dot(g, wf2[...]) + bf2[...])
    rows = lax.broadcasted_iota(jnp.int32, (NP, 1), 0)
    valid = (rows < N).astype(jnp.float32)
    h_ref[:, 0:1] = hp * valid
    h_ref[:, 1:2] = hf * valid
    h_ref[:, 2:3] = valid


def _mlp(x_part, x_family, wp1, bp1, wp2, bp2, wp3, bp3, wf1, bf1, wf2, bf2):
    xp = jnp.zeros((NP, x_part.shape[1]), jnp.float32).at[:N].set(x_part)
    xf = jnp.zeros((NP, x_family.shape[1]), jnp.float32).at[:N].set(x_family)
    return pl.pallas_call(
        _mlp_body,
        out_shape=jax.ShapeDtypeStruct((NP, 3), jnp.float32),
    )(xp, xf, wp1, bp1.reshape(1, -1), wp2, bp2.reshape(1, -1), wp3,
      bp3.reshape(1, -1), wf1, bf1.reshape(1, -1), wf2, bf2.reshape(1, -1))


BR = 256
GRID = NP // BR


def _rgcn_body(din, dout, b_ref, he_ref, wrel0, wrel1, wroot, bias, out_ref):
    i = pl.program_id(0)
    dot = functools.partial(jnp.dot, preferred_element_type=jnp.float32)
    rows = i * BR + lax.broadcasted_iota(jnp.int32, (BR, NP), 0)
    cols = lax.broadcasted_iota(jnp.int32, (BR, NP), 1)
    bblk = jnp.where(rows == cols, 0.0, b_ref[...])
    se = dot(bblk, he_ref[...])
    s1 = se[:, 0:din]
    deg = se[:, din:din + 1]
    hblk = he_ref[pl.ds(i * BR, BR), 0:din]
    total = jnp.sum(he_ref[:, 0:din], axis=0, keepdims=True)
    agg0 = total - hblk - s1
    cnt0 = jnp.maximum((N - 1.0) - deg, 1.0)
    cnt1 = jnp.maximum(deg, 1.0)
    out = (dot(hblk, wroot[...]) + bias[...]
           + dot(agg0 / cnt0, wrel0[...])
           + dot(s1 / cnt1, wrel1[...]))
    out = jax.nn.relu(out)
    rmask = (i * BR + lax.broadcasted_iota(jnp.int32, (BR, 1), 0)) < N
    out = jnp.where(rmask, out, 0.0)
    out_ref[:, 0:dout] = out
    if dout < out_ref.shape[1]:
        out_ref[:, dout:dout + 1] = rmask.astype(jnp.float32)


def _rgcn_layer(bmat, he, din, dout, wrel, wroot, bias, with_ones):
    owid = dout + 1 if with_ones else dout
    full = lambda a: pl.BlockSpec(a.shape, lambda i: (0,) * a.ndim)
    return pl.pallas_call(
        functools.partial(_rgcn_body, din, dout),
        grid=(GRID,),
        in_specs=[
            pl.BlockSpec((BR, NP), lambda i: (i, 0)),
            full(he),
            pl.BlockSpec((din, dout), lambda i: (0, 0)),
            pl.BlockSpec((din, dout), lambda i: (0, 0)),
            pl.BlockSpec((din, dout), lambda i: (0, 0)),
            pl.BlockSpec((1, dout), lambda i: (0, 0)),
        ],
        out_specs=pl.BlockSpec((BR, owid), lambda i: (i, 0)),
        out_shape=jax.ShapeDtypeStruct((NP, owid), jnp.float32),
    )(bmat, he, wrel[0], wrel[1], wroot, bias.reshape(1, -1))


def kernel(x_part, x_family, edges, Wp1, bp1, Wp2, bp2, Wp3, bp3,
           Wf1, bf1, Wf2, bf2, Wrel1, Wroot1, b1, Wrel2, Wroot2, b2):
    bmat = _build_adjacency(edges)
    h1e = _mlp(x_part, x_family, Wp1, bp1, Wp2, bp2, Wp3, bp3,
               Wf1, bf1, Wf2, bf2)
    h2e = _rgcn_layer(bmat, h1e, 2, 8, Wrel1, Wroot1, b1, with_ones=True)
    h3 = _rgcn_layer(bmat, h2e, 8, 16, Wrel2, Wroot2, b2, with_ones=False)
    return h3[:N]


# speedup vs baseline: 1.7677x; 1.4996x over previous
import functools

import jax
import jax.numpy as jnp
from jax import lax
from jax.experimental import pallas as pl
from jax.experimental.pallas import tpu as pltpu
from jax.experimental.pallas import tpu_sc as plsc

N = 2000
NP = 2048
EP = 32768
NW = 32
EPW = EP // NW
NDMA = 2 * EPW // 128


WROWS = 512
WINW = WROWS * NP
NWIN = 2
OSL = WINW // 16
EPS = EP // 16
ZN = 16384


def _sc_build_body(pk_hbm, b_out, shared, pk_v, idx_v, vals_v, zbuf, sem):
    c = lax.axis_index("c")
    sid = lax.axis_index("s")

    @pl.loop(0, ZN // 16)
    def _zero(i):
        zbuf[pl.ds(i * 16, 16)] = jnp.zeros((16,), jnp.float32)

    for k in range(8):
        vals_v[pl.ds(k * 16, 16)] = jnp.full((16,), 1.0, jnp.float32)
    zcopies = [
        pltpu.make_async_copy(zbuf, shared.at[pl.ds(sid * OSL + k * ZN, ZN)],
                              sem)
        for k in range(OSL // ZN)
    ]
    for cp in zcopies:
        cp.start()
    pltpu.sync_copy(pk_hbm.at[pl.ds(sid * EPS, EPS)], pk_v)
    for cp in zcopies:
        cp.wait()

    for w in range(NWIN):
        base = (c * NWIN + w) * WINW
        for i in range(EPS // 16):
            p = pk_v[pl.ds(i * 16, 16)]
            s = lax.shift_right_logical(p, 11)
            d = p & (NP - 1)
            for k, a in enumerate((p, d * NP + s)):
                loc = a - base
                ok = (loc >= 0) & (loc < WINW)
                dump = (jnp.arange(16, dtype=jnp.int32) * NP
                        + (N + (2 * i + k) % (NP - N)))
                r, col = divmod((k * EPS + i * 16), 128)
                idx_v[r, pl.ds(col, 16)] = jnp.where(ok, loc, dump)
        plsc.subcore_barrier()
        copies = [
            pltpu.make_async_copy(vals_v, shared.at[idx_v.at[j]], sem)
            for j in range(2 * EPS // 128)
        ]
        for cp in copies:
            cp.start()
        for cp in copies:
            cp.wait()
        plsc.subcore_barrier()
        pltpu.sync_copy(shared.at[pl.ds(sid * OSL, OSL)],
                        b_out.at[pl.ds(base + sid * OSL, OSL)])
        if w + 1 < NWIN:
            rz = [
                pltpu.make_async_copy(
                    zbuf, shared.at[pl.ds(sid * OSL + k * ZN, ZN)], sem)
                for k in range(OSL // ZN)
            ]
            for cp in rz:
                cp.start()
            for cp in rz:
                cp.wait()


def _build_adjacency(edges):
    npad = EP - edges.shape[1]
    pad_src = jnp.arange(npad, dtype=jnp.int32) % N
    src = jnp.concatenate([edges[0], pad_src])
    dst = jnp.concatenate([edges[1], jnp.full((npad,), NP - 1, jnp.int32)])
    pk = src * NP + dst
    mesh = plsc.VectorSubcoreMesh(core_axis_name="c", subcore_axis_name="s")
    sc = pl.kernel(
        _sc_build_body,
        out_type=jax.ShapeDtypeStruct((NP * NP,), jnp.float32),
        mesh=mesh,
        scratch_types=[
            pltpu.VMEM_SHARED((WINW,), jnp.float32),
            pltpu.VMEM((EPS,), jnp.int32),
            pltpu.VMEM((2 * EPS // 128, 128), jnp.int32),
            pltpu.VMEM((128,), jnp.float32),
            pltpu.VMEM((ZN,), jnp.float32),
            pltpu.SemaphoreType.DMA,
        ],
    )
    return sc(pk).reshape(NP, NP)


def _mlp_body(xp_ref, xf_ref, wp1, bp1, wp2, bp2, wp3, bp3, wf1, bf1, wf2, bf2,
              h_ref):
    dot = functools.partial(jnp.dot, preferred_element_type=jnp.float32)
    h = jax.nn.relu(dot(xp_ref[...], wp1[...]) + bp1[...])
    h = jax.nn.relu(dot(h, wp2[...]) + bp2[...])
    hp = jax.nn.relu(dot(h, wp3[...]) + bp3[...])
    g = jax.nn.relu(dot(xf_ref[...], wf1[...]) + bf1[...])
    hf = jax.nn.relu(dot(g, wf2[...]) + bf2[...])
    rows = lax.broadcasted_iota(jnp.int32, (NP, 1), 0)
    valid = (rows < N).astype(jnp.float32)
    h_ref[:, 0:1] = hp * valid
    h_ref[:, 1:2] = hf * valid
    h_ref[:, 2:3] = valid


def _mlp(x_part, x_family, wp1, bp1, wp2, bp2, wp3, bp3, wf1, bf1, wf2, bf2):
    xp = jnp.zeros((NP, x_part.shape[1]), jnp.float32).at[:N].set(x_part)
    xf = jnp.zeros((NP, x_family.shape[1]), jnp.float32).at[:N].set(x_family)
    return pl.pallas_call(
        _mlp_body,
        out_shape=jax.ShapeDtypeStruct((NP, 3), jnp.float32),
    )(xp, xf, wp1, bp1.reshape(1, -1), wp2, bp2.reshape(1, -1), wp3,
      bp3.reshape(1, -1), wf1, bf1.reshape(1, -1), wf2, bf2.reshape(1, -1))


BR = 256
GRID = NP // BR


def _rgcn_body(din, dout, b_ref, he_ref, wrel0, wrel1, wroot, bias, out_ref):
    i = pl.program_id(0)
    dot = functools.partial(jnp.dot, preferred_element_type=jnp.float32)
    rows = i * BR + lax.broadcasted_iota(jnp.int32, (BR, NP), 0)
    cols = lax.broadcasted_iota(jnp.int32, (BR, NP), 1)
    bblk = jnp.where(rows == cols, 0.0, b_ref[...])
    se = dot(bblk, he_ref[...])
    s1 = se[:, 0:din]
    deg = se[:, din:din + 1]
    hblk = he_ref[pl.ds(i * BR, BR), 0:din]
    total = jnp.sum(he_ref[:, 0:din], axis=0, keepdims=True)
    agg0 = total - hblk - s1
    cnt0 = jnp.maximum((N - 1.0) - deg, 1.0)
    cnt1 = jnp.maximum(deg, 1.0)
    out = (dot(hblk, wroot[...]) + bias[...]
           + dot(agg0 / cnt0, wrel0[...])
           + dot(s1 / cnt1, wrel1[...]))
    out = jax.nn.relu(out)
    rmask = (i * BR + lax.broadcasted_iota(jnp.int32, (BR, 1), 0)) < N
    out = jnp.where(rmask, out, 0.0)
    out_ref[:, 0:dout] = out
    if dout < out_ref.shape[1]:
        out_ref[:, dout:dout + 1] = rmask.astype(jnp.float32)


def _rgcn_layer(bmat, he, din, dout, wrel, wroot, bias, with_ones):
    owid = dout + 1 if with_ones else dout
    full = lambda a: pl.BlockSpec(a.shape, lambda i: (0,) * a.ndim)
    return pl.pallas_call(
        functools.partial(_rgcn_body, din, dout),
        grid=(GRID,),
        in_specs=[
            pl.BlockSpec((BR, NP), lambda i: (i, 0)),
            full(he),
            pl.BlockSpec((din, dout), lambda i: (0, 0)),
            pl.BlockSpec((din, dout), lambda i: (0, 0)),
            pl.BlockSpec((din, dout), lambda i: (0, 0)),
            pl.BlockSpec((1, dout), lambda i: (0, 0)),
        ],
        out_specs=pl.BlockSpec((BR, owid), lambda i: (i, 0)),
        out_shape=jax.ShapeDtypeStruct((NP, owid), jnp.float32),
    )(bmat, he, wrel[0], wrel[1], wroot, bias.reshape(1, -1))


def kernel(x_part, x_family, edges, Wp1, bp1, Wp2, bp2, Wp3, bp3,
           Wf1, bf1, Wf2, bf2, Wrel1, Wroot1, b1, Wrel2, Wroot2, b2):
    bmat = _build_adjacency(edges)
    h1e = _mlp(x_part, x_family, Wp1, bp1, Wp2, bp2, Wp3, bp3,
               Wf1, bf1, Wf2, bf2)
    h2e = _rgcn_layer(bmat, h1e, 2, 8, Wrel1, Wroot1, b1, with_ones=True)
    h3 = _rgcn_layer(bmat, h2e, 8, 16, Wrel2, Wroot2, b2, with_ones=False)
    return h3[:N]
